# baseline (speedup 1.0000x reference)
import functools

import jax
import jax.numpy as jnp
from jax import lax
from jax.experimental import pallas as pl
from jax.experimental.pallas import tpu as pltpu

_MARGIN = 0.25
_GAMMA = 64.0
_B = 1024
_N = 100000
_H = 2000
_NBLK = _N // _H
_CH = 8
_LOG2E = 1.4426950408889634
_LN2 = 0.6931471805599453
_C0 = _GAMMA * _MARGIN * _MARGIN * _LOG2E
_CS = 9.60897927029168
_NEG = -1e30


def _body(tgt_ref, feat_ref, out_ref, m_ref, s_ref, tv_ref):
    c = pl.program_id(0)

    @pl.when(c == 0)
    def _init():
        m_ref[...] = jnp.zeros((_CH, _B), jnp.float32)
        s_ref[...] = jnp.zeros((_CH, _B), jnp.float32)
        tv_ref[...] = jnp.zeros((_CH, _B), jnp.float32)

    iota0 = lax.broadcasted_iota(jnp.int32, (_CH, _B), 0)
    tsh = jnp.broadcast_to(tgt_ref[...] - c * _H, (_CH, _B))

    def step(k, carry):
        m, s, tv = carry
        x = feat_ref[pl.ds(pl.multiple_of(k * _CH, _CH), _CH), :]
        is_t = (iota0 + k * _CH) == tsh
        y = jnp.maximum(x, -_MARGIN) * _CS
        h = jnp.where(is_t, _NEG, y * y)
        tv = tv + jnp.where(is_t, x, 0.0)
        m_new = jnp.maximum(m, h)
        s = s * jnp.exp2(m - m_new) + jnp.exp2(h - m_new)
        return (m_new, s, tv)

    carry0 = (m_ref[...], s_ref[...], tv_ref[...])
    m1, s1, tv1 = lax.fori_loop(0, _H // _CH, step, carry0)
    m_ref[...] = m1
    s_ref[...] = s1
    tv_ref[...] = tv1

    @pl.when(c == _NBLK - 1)
    def _fin():
        m8 = m_ref[...]
        s8 = s_ref[...]
        mstar = jnp.max(m8, axis=0, keepdims=True)
        sstar = jnp.sum(s8 * jnp.exp2(m8 - mstar), axis=0, keepdims=True)
        tv = jnp.sum(tv_ref[...], axis=0, keepdims=True)
        fin_t = jnp.maximum(1.0 + _MARGIN - tv, 0.0) * ((tv - (1.0 - _MARGIN)) * _GAMMA)
        h_t = fin_t * _LOG2E + _C0
        big = jnp.maximum(mstar, h_t)
        tot = sstar * jnp.exp2(mstar - big) + jnp.exp2(h_t - big)
        lse = (big - _C0 + jnp.log2(tot)) * _LN2
        out_ref[...] = jnp.mean(lse - fin_t).reshape(1, 1)


@functools.partial(jax.jit, static_argnames=("interpret",))
def kernel(feat, target, interpret=False):
    tgt = target.astype(jnp.int32).reshape(1, _B)
    feat_t = feat.T
    out = pl.pallas_call(
        _body,
        grid=(_NBLK,),
        in_specs=[
            pl.BlockSpec((1, _B), lambda c: (0, 0)),
            pl.BlockSpec((_H, _B), lambda c: (c, 0)),
        ],
        out_specs=pl.BlockSpec((1, 1), lambda c: (0, 0)),
        out_shape=jax.ShapeDtypeStruct((1, 1), jnp.float32),
        scratch_shapes=[
            pltpu.VMEM((_CH, _B), jnp.float32),
            pltpu.VMEM((_CH, _B), jnp.float32),
            pltpu.VMEM((_CH, _B), jnp.float32),
        ],
        interpret=interpret,
    )(tgt, feat_t)
    return out[0, 0]


# speedup vs baseline: 1.4003x; 1.4003x over previous
import functools

import jax
import jax.numpy as jnp
from jax import lax
from jax.experimental import pallas as pl
from jax.experimental.pallas import tpu as pltpu

_MARGIN = 0.25
_GAMMA = 64.0
_B = 1024
_N = 100000
_H = 2000
_NBLK = _N // _H
_LOG2E = 1.4426950408889634
_LN2 = 0.6931471805599453
_C0 = _GAMMA * _MARGIN * _MARGIN * _LOG2E
_CS = 9.60897927029168
_NEG = -1e30


def _body(tgt_ref, feat_ref, out_ref, m_ref, s_ref, tv_ref):
    c = pl.program_id(0)

    @pl.when(c == 0)
    def _init():
        m_ref[...] = jnp.zeros((1, _B), jnp.float32)
        s_ref[...] = jnp.zeros((1, _B), jnp.float32)
        tv_ref[...] = jnp.zeros((1, _B), jnp.float32)

    x = feat_ref[...]
    iota = lax.broadcasted_iota(jnp.int32, (_H, _B), 0)
    tsh = tgt_ref[...] - c * _H
    is_t = iota == tsh
    y = jnp.maximum(x, -_MARGIN) * _CS
    h = jnp.where(is_t, _NEG, y * y)
    tv_ref[...] += jnp.sum(jnp.where(is_t, x, 0.0), axis=0, keepdims=True)
    bmax = jnp.max(h, axis=0, keepdims=True)
    m_old = m_ref[...]
    m_new = jnp.maximum(m_old, bmax)
    s_ref[...] = s_ref[...] * jnp.exp2(m_old - m_new) + jnp.sum(
        jnp.exp2(h - m_new), axis=0, keepdims=True
    )
    m_ref[...] = m_new

    @pl.when(c == _NBLK - 1)
    def _fin():
        tv = tv_ref[...]
        fin_t = jnp.maximum(1.0 + _MARGIN - tv, 0.0) * ((tv - (1.0 - _MARGIN)) * _GAMMA)
        h_t = fin_t * _LOG2E + _C0
        m = m_ref[...]
        s = s_ref[...]
        big = jnp.maximum(m, h_t)
        tot = s * jnp.exp2(m - big) + jnp.exp2(h_t - big)
        lse = (big - _C0 + jnp.log2(tot)) * _LN2
        out_ref[...] = jnp.mean(lse - fin_t).reshape(1, 1)


@functools.partial(jax.jit, static_argnames=("interpret",))
def kernel(feat, target, interpret=False):
    tgt = target.astype(jnp.int32).reshape(1, _B)
    feat_t = feat.T
    out = pl.pallas_call(
        _body,
        grid=(_NBLK,),
        in_specs=[
            pl.BlockSpec((1, _B), lambda c: (0, 0)),
            pl.BlockSpec((_H, _B), lambda c: (c, 0)),
        ],
        out_specs=pl.BlockSpec((1, 1), lambda c: (0, 0)),
        out_shape=jax.ShapeDtypeStruct((1, 1), jnp.float32),
        scratch_shapes=[
            pltpu.VMEM((1, _B), jnp.float32),
            pltpu.VMEM((1, _B), jnp.float32),
            pltpu.VMEM((1, _B), jnp.float32),
        ],
        interpret=interpret,
    )(tgt, feat_t)
    return out[0, 0]
